# baseline (speedup 1.0000x reference)
import jax
import jax.numpy as jnp
from jax.experimental import pallas as pl

_BLOCK_T = 512


def _fused_kernel(q_ref, items_ref, o_ref):
    s = jax.lax.dot_general(
        q_ref[...], items_ref[...],
        (((1,), (1,)), ((), ())),
        preferred_element_type=jnp.float32,
    )
    m = jnp.max(s, axis=1, keepdims=True)
    denom = jnp.sum(jnp.exp(s - m), axis=1, keepdims=True)
    o_ref[...] = 1.0 / denom


@jax.jit
def kernel(queries, items):
    d_model = queries.shape[-1]
    q = queries.reshape(-1, d_model)
    t = q.shape[0]
    n_items = items.shape[0]
    grid = (t // _BLOCK_T,)
    out = pl.pallas_call(
        _fused_kernel,
        grid=grid,
        in_specs=[
            pl.BlockSpec((_BLOCK_T, d_model), lambda i: (i, 0)),
            pl.BlockSpec((n_items, d_model), lambda i: (0, 0)),
        ],
        out_specs=pl.BlockSpec((_BLOCK_T, 1), lambda i: (i, 0)),
        out_shape=jax.ShapeDtypeStruct((t, 1), jnp.float32),
    )(q, items)
    return out


# speedup vs baseline: 1.3907x; 1.3907x over previous
import jax
import jax.numpy as jnp
from jax.experimental import pallas as pl

_BLOCK_N = 32


def _fused_kernel(q_ref, items_ref, o_ref):
    b, l, c = q_ref.shape
    s = jax.lax.dot_general(
        q_ref[...], items_ref[...],
        (((2,), (1,)), ((), ())),
        preferred_element_type=jnp.float32,
    )
    m = jnp.max(s, axis=2, keepdims=True)
    denom = jnp.sum(jnp.exp(s - m), axis=2, keepdims=True)
    o_ref[...] = (1.0 / denom).reshape(b * l, 1)


@jax.jit
def kernel(queries, items):
    n, l, c = queries.shape
    m_items = items.shape[0]
    grid = (n // _BLOCK_N,)
    out = pl.pallas_call(
        _fused_kernel,
        grid=grid,
        in_specs=[
            pl.BlockSpec((_BLOCK_N, l, c), lambda i: (i, 0, 0)),
            pl.BlockSpec((m_items, c), lambda i: (0, 0)),
        ],
        out_specs=pl.BlockSpec((_BLOCK_N * l, 1), lambda i: (i, 0)),
        out_shape=jax.ShapeDtypeStruct((n * l, 1), jnp.float32),
    )(queries, items)
    return out


# speedup vs baseline: 1.8865x; 1.3565x over previous
import jax
import jax.numpy as jnp
from jax.experimental import pallas as pl

_BLOCK_N = 128


def _fused_kernel(q_ref, items_ref, o_ref):
    b, l, c = q_ref.shape
    q = q_ref[...].reshape(b * l, c)
    s = jax.lax.dot_general(
        q, items_ref[...],
        (((1,), (1,)), ((), ())),
        preferred_element_type=jnp.float32,
    )
    m = jnp.max(s, axis=1, keepdims=True)
    denom = jnp.sum(jnp.exp(s - m), axis=1, keepdims=True)
    o_ref[...] = 1.0 / denom


@jax.jit
def kernel(queries, items):
    n, l, c = queries.shape
    m_items = items.shape[0]
    grid = (n // _BLOCK_N,)
    out = pl.pallas_call(
        _fused_kernel,
        grid=grid,
        in_specs=[
            pl.BlockSpec((_BLOCK_N, l, c), lambda i: (i, 0, 0)),
            pl.BlockSpec((m_items, c), lambda i: (0, 0)),
        ],
        out_specs=pl.BlockSpec((_BLOCK_N * l, 1), lambda i: (i, 0)),
        out_shape=jax.ShapeDtypeStruct((n * l, 1), jnp.float32),
    )(queries, items)
    return out


# speedup vs baseline: 3.5591x; 1.8866x over previous
import jax
import jax.numpy as jnp
from jax.experimental import pallas as pl

_BLOCK_N = 128
_LOG2E = 1.4426950408889634


def _fused_kernel(q_ref, items_ref, o_ref):
    l, b, c = q_ref.shape
    q = q_ref[...] * _LOG2E
    s = jax.lax.dot_general(
        q, items_ref[...],
        (((2,), (1,)), ((), ())),
        preferred_element_type=jnp.float32,
    )
    m = jnp.max(s, axis=2, keepdims=True)
    denom = jnp.sum(jnp.exp2(s - m), axis=2)
    o_ref[...] = 1.0 / denom


@jax.jit
def kernel(queries, items):
    n, l, c = queries.shape
    m_items = items.shape[0]
    t = n * l
    qt = jnp.transpose(queries, (1, 0, 2))
    grid = (n // _BLOCK_N,)
    rows_per_step = _BLOCK_N * l // 128
    out = pl.pallas_call(
        _fused_kernel,
        grid=grid,
        in_specs=[
            pl.BlockSpec((l, _BLOCK_N, c), lambda i: (0, i, 0)),
            pl.BlockSpec((m_items, c), lambda i: (0, 0)),
        ],
        out_specs=pl.BlockSpec((l, _BLOCK_N), lambda i: (0, i)),
        out_shape=jax.ShapeDtypeStruct((l, n), jnp.float32),
    )(qt, items)
    return out.T.reshape(t, 1)
